# baseline (speedup 1.0000x reference)
import jax
import jax.numpy as jnp
from jax import lax
from jax.experimental import pallas as pl
from jax.experimental.pallas import tpu as pltpu

_S = 2
_S3 = _S ** 3
_D = 128
_B = 2000
_N = 50000
_NG = _N // _B


def _body(scale_ref, coords_ref, feats_ref, coords_out_ref, feats_out_ref,
          obuf, sem):
    i = pl.program_id(0)
    slot = lax.rem(i, 2)

    @pl.when(i >= 2)
    def _wait_prev():
        pltpu.make_async_copy(
            obuf.at[slot], feats_out_ref.at[pl.ds((i - 2) * _B, _B)],
            sem.at[slot]).wait()

    f = feats_ref[...]
    obuf[slot] = jnp.broadcast_to(f[:, None, :], (_B, _S3, _D))
    pltpu.make_async_copy(
        obuf.at[slot], feats_out_ref.at[pl.ds(i * _B, _B)],
        sem.at[slot]).start()

    c = coords_ref[...]
    s = scale_ref[0]
    c_rep = lax.broadcast_in_dim(c, (_B, _S3, 4), (0, 2)).reshape(_B * _S3, 4)
    r = lax.broadcasted_iota(jnp.int32, (_B * _S3, 4), 0)
    k = lax.broadcasted_iota(jnp.int32, (_B * _S3, 4), 1)
    j = r & 7
    mult = jnp.where(k == 0, 1, s)
    off = jnp.where(
        k == 0, 0,
        jnp.where(k == 1, (j >> 2) & 1,
                  jnp.where(k == 2, (j >> 1) & 1, j & 1)))
    coords_out_ref[...] = c_rep * mult + off

    @pl.when(i == _NG - 1)
    def _drain():
        pltpu.make_async_copy(
            obuf.at[1 - slot], feats_out_ref.at[pl.ds((i - 1) * _B, _B)],
            sem.at[1 - slot]).wait()
        pltpu.make_async_copy(
            obuf.at[slot], feats_out_ref.at[pl.ds(i * _B, _B)],
            sem.at[slot]).wait()


def kernel(coords, feats, scale):
    N, d = feats.shape
    grid = (_NG,)
    scale_arr = jnp.asarray(scale, jnp.int32).reshape(1)
    coords_out, feats_out = pl.pallas_call(
        _body,
        grid=grid,
        in_specs=[
            pl.BlockSpec(memory_space=pltpu.SMEM),
            pl.BlockSpec((_B, 4), lambda i: (i, 0)),
            pl.BlockSpec((_B, d), lambda i: (i, 0)),
        ],
        out_specs=[
            pl.BlockSpec((_B * _S3, 4), lambda i: (i, 0)),
            pl.BlockSpec(memory_space=pl.ANY),
        ],
        out_shape=[
            jax.ShapeDtypeStruct((N * _S3, 4), jnp.int32),
            jax.ShapeDtypeStruct((N, _S3, d), jnp.float32),
        ],
        scratch_shapes=[
            pltpu.VMEM((2, _B, _S3, _D), jnp.float32),
            pltpu.SemaphoreType.DMA((2,)),
        ],
    )(scale_arr, coords, feats)
    return coords_out, feats_out.reshape(N * _S3, d)
